# baseline (speedup 1.0000x reference)
import jax
import jax.numpy as jnp
from jax import lax
from jax.experimental import pallas as pl
from jax.experimental.pallas import tpu as pltpu
from jax.experimental.pallas import tpu_sc as plsc


NC, NS = 2, 16
NW = NC * NS
L = 16
NQ = 4
Q = 192 // NQ


def _sc_add_body(x_hbm, t_hbm, o_hbm, table_v, b0, b1, b2, b3,
                 si0, si1, si2, si3, so0, so1, so2, so3):
    wid = lax.axis_index("s") * NC + lax.axis_index("c")
    rows = x_hbm.shape[0] // NW
    base = wid * rows
    embed, seq = t_hbm.shape
    n = rows * NQ
    bufs = (b0, b1, b2, b3)
    sins = (si0, si1, si2, si3)
    souts = (so0, so1, so2, so3)
    pltpu.sync_copy(t_hbm, table_v)

    def src(g):
        return x_hbm.at[base + g // NQ, pl.ds((g % NQ) * Q, Q)]

    def dst(g):
        return o_hbm.at[base + g // NQ, pl.ds((g % NQ) * Q, Q)]

    for h in range(2):
        pltpu.async_copy(src(h), bufs[h], sins[h])

    def row_body(i, carry):
        for j in range(NQ):
            g = NQ * i + j
            buf, sin, sout = bufs[j], sins[j], souts[j]
            pltpu.make_async_copy(src(g), buf, sin).wait()

            def step(e, c):
                for u in range(seq // L):
                    o = u * L
                    buf[e, pl.ds(o, L)] = (
                        buf[e, pl.ds(o, L)]
                        + table_v[j * Q + e, pl.ds(o, L)]
                    )
                return c

            lax.fori_loop(0, Q, step, 0)
            pltpu.async_copy(buf, dst(g), sout)

            k = (j + 2) % NQ

            @pl.when(g >= 2)
            def _():
                pltpu.make_async_copy(bufs[k], dst(g - 2), souts[k]).wait()

            @pl.when(g + 2 < n)
            def _():
                pltpu.async_copy(src(g + 2), bufs[k], sins[k])

        return carry

    lax.fori_loop(0, rows, row_body, 0)
    for g in (n - 2, n - 1):
        k = g % NQ
        pltpu.make_async_copy(bufs[k], dst(g), souts[k]).wait()


def kernel(x, pos_table):
    batch, seq_len, embed = x.shape
    xt = jnp.transpose(x, (0, 2, 1))
    tt = jnp.transpose(pos_table[:seq_len], (1, 0))
    mesh = plsc.VectorSubcoreMesh(core_axis_name="c", subcore_axis_name="s")
    out_t = pl.kernel(
        _sc_add_body,
        out_type=jax.ShapeDtypeStruct(xt.shape, x.dtype),
        mesh=mesh,
        scratch_types=[
            pltpu.VMEM((embed, seq_len), jnp.float32),
            pltpu.VMEM((Q, seq_len), jnp.float32),
            pltpu.VMEM((Q, seq_len), jnp.float32),
            pltpu.VMEM((Q, seq_len), jnp.float32),
            pltpu.VMEM((Q, seq_len), jnp.float32),
            pltpu.SemaphoreType.DMA,
            pltpu.SemaphoreType.DMA,
            pltpu.SemaphoreType.DMA,
            pltpu.SemaphoreType.DMA,
            pltpu.SemaphoreType.DMA,
            pltpu.SemaphoreType.DMA,
            pltpu.SemaphoreType.DMA,
            pltpu.SemaphoreType.DMA,
        ],
        compiler_params=pltpu.CompilerParams(use_tc_tiling_on_sc=True),
    )(xt, tt)
    return jnp.transpose(out_t, (0, 2, 1))


# speedup vs baseline: 2.2623x; 2.2623x over previous
import jax
import jax.numpy as jnp
from jax import lax
from jax.experimental import pallas as pl
from jax.experimental.pallas import tpu as pltpu
from jax.experimental.pallas import tpu_sc as plsc


NC, NS = 2, 16
L = 16
BATCH_BLOCK = 64


def _sc_gather_body(t_hbm, o_hbm, rows_v, sem):
    wid = lax.axis_index("s") * NC + lax.axis_index("c")
    n_pos = o_hbm.shape[0]
    active = n_pos // L

    @pl.when(wid < active)
    def _():
        for k in range(L):
            pltpu.async_copy(t_hbm.at[wid * L + k], rows_v.at[k], sem)
        for k in range(L):
            pltpu.make_async_copy(t_hbm.at[wid * L + k], rows_v.at[k], sem).wait()
        pltpu.sync_copy(rows_v, o_hbm.at[pl.ds(wid * L, L)])


def _tc_add_body(x_ref, t_ref, o_ref):
    o_ref[...] = x_ref[...] + t_ref[...][None, :, :]


def kernel(x, pos_table):
    batch, seq_len, embed = x.shape

    mesh = plsc.VectorSubcoreMesh(core_axis_name="c", subcore_axis_name="s")
    pos_emb = pl.kernel(
        _sc_gather_body,
        out_type=jax.ShapeDtypeStruct((seq_len, embed), pos_table.dtype),
        mesh=mesh,
        scratch_types=[
            pltpu.VMEM((L, embed), jnp.float32),
            pltpu.SemaphoreType.DMA,
        ],
    )(pos_table)

    xt = jnp.transpose(x, (0, 2, 1))
    pe_t = jnp.transpose(pos_emb, (1, 0))
    out_t = pl.pallas_call(
        _tc_add_body,
        grid=(batch // BATCH_BLOCK,),
        in_specs=[
            pl.BlockSpec((BATCH_BLOCK, embed, seq_len), lambda i: (i, 0, 0)),
            pl.BlockSpec((embed, seq_len), lambda i: (0, 0)),
        ],
        out_specs=pl.BlockSpec((BATCH_BLOCK, embed, seq_len), lambda i: (i, 0, 0)),
        out_shape=jax.ShapeDtypeStruct((batch, embed, seq_len), x.dtype),
    )(xt, pe_t)
    return jnp.transpose(out_t, (0, 2, 1))


# speedup vs baseline: 2.2700x; 1.0034x over previous
import jax
import jax.numpy as jnp
from jax import lax
from jax.experimental import pallas as pl
from jax.experimental.pallas import tpu as pltpu
from jax.experimental.pallas import tpu_sc as plsc


NC, NS = 2, 16
L = 16
BATCH_BLOCK = 64


def _sc_gather_body(t_hbm, o_hbm, rows_v, sem):
    wid = lax.axis_index("s") * NC + lax.axis_index("c")
    n_pos = o_hbm.shape[0]
    active = n_pos // L

    @pl.when(wid < active)
    def _():
        for k in range(2):
            pltpu.async_copy(
                t_hbm.at[pl.ds((wid * 2 + k) * 8, 8)],
                rows_v.at[pl.ds(k * 8, 8)],
                sem,
            )
        for k in range(2):
            pltpu.make_async_copy(
                t_hbm.at[pl.ds((wid * 2 + k) * 8, 8)],
                rows_v.at[pl.ds(k * 8, 8)],
                sem,
            ).wait()
        pltpu.sync_copy(rows_v, o_hbm.at[pl.ds(wid * L, L)])


def _tc_add_body(x_ref, t_ref, o_ref):
    o_ref[...] = x_ref[...] + t_ref[...][None, :, :]


def kernel(x, pos_table):
    batch, seq_len, embed = x.shape

    mesh = plsc.VectorSubcoreMesh(core_axis_name="c", subcore_axis_name="s")
    pos_emb = pl.kernel(
        _sc_gather_body,
        out_type=jax.ShapeDtypeStruct((seq_len, embed), pos_table.dtype),
        mesh=mesh,
        scratch_types=[
            pltpu.VMEM((L, embed), jnp.float32),
            pltpu.SemaphoreType.DMA,
        ],
        compiler_params=pltpu.CompilerParams(use_tc_tiling_on_sc=True),
    )(pos_table)

    xt = jnp.transpose(x, (0, 2, 1))
    pe_t = jnp.transpose(pos_emb, (1, 0))
    out_t = pl.pallas_call(
        _tc_add_body,
        grid=(batch // BATCH_BLOCK,),
        in_specs=[
            pl.BlockSpec((BATCH_BLOCK, embed, seq_len), lambda i: (i, 0, 0)),
            pl.BlockSpec((embed, seq_len), lambda i: (0, 0)),
        ],
        out_specs=pl.BlockSpec((BATCH_BLOCK, embed, seq_len), lambda i: (i, 0, 0)),
        out_shape=jax.ShapeDtypeStruct((batch, embed, seq_len), x.dtype),
    )(xt, pe_t)
    return jnp.transpose(out_t, (0, 2, 1))


# speedup vs baseline: 2.2841x; 1.0062x over previous
import jax
import jax.numpy as jnp
from jax import lax
from jax.experimental import pallas as pl
from jax.experimental.pallas import tpu as pltpu
from jax.experimental.pallas import tpu_sc as plsc


NC, NS = 2, 16
L = 16
BATCH_BLOCK = 64
TAIL_BLOCKS = 1


def _sc_gather_body(t_hbm, o_hbm, rows_v, sem):
    wid = lax.axis_index("s") * NC + lax.axis_index("c")
    n_pos = o_hbm.shape[0]
    active = n_pos // L

    @pl.when(wid < active)
    def _():
        for k in range(2):
            pltpu.async_copy(
                t_hbm.at[pl.ds((wid * 2 + k) * 8, 8)],
                rows_v.at[pl.ds(k * 8, 8)],
                sem,
            )
        for k in range(2):
            pltpu.make_async_copy(
                t_hbm.at[pl.ds((wid * 2 + k) * 8, 8)],
                rows_v.at[pl.ds(k * 8, 8)],
                sem,
            ).wait()
        pltpu.sync_copy(rows_v, o_hbm.at[pl.ds(wid * L, L)])


def _tc_add_body(x_ref, t_ref, o_ref):
    o_ref[...] = x_ref[...] + t_ref[...][None, :, :]


def _tc_add_tail_body(prev_ref, x_ref, t_ref, o_ref):
    del prev_ref
    o_ref[...] = x_ref[...] + t_ref[...][None, :, :]


def kernel(x, pos_table):
    batch, seq_len, embed = x.shape
    nblocks = batch // BATCH_BLOCK
    head = nblocks - TAIL_BLOCKS

    mesh = plsc.VectorSubcoreMesh(core_axis_name="c", subcore_axis_name="s")
    pos_emb = pl.kernel(
        _sc_gather_body,
        out_type=jax.ShapeDtypeStruct((seq_len, embed), pos_table.dtype),
        mesh=mesh,
        scratch_types=[
            pltpu.VMEM((L, embed), jnp.float32),
            pltpu.SemaphoreType.DMA,
        ],
        compiler_params=pltpu.CompilerParams(use_tc_tiling_on_sc=True),
    )(pos_table)

    xt = jnp.transpose(x, (0, 2, 1))
    tt = jnp.transpose(pos_table[:seq_len], (1, 0))
    pe_t = jnp.transpose(pos_emb, (1, 0))

    head_out = pl.pallas_call(
        _tc_add_body,
        grid=(head,),
        in_specs=[
            pl.BlockSpec((BATCH_BLOCK, embed, seq_len), lambda i: (i, 0, 0)),
            pl.BlockSpec((embed, seq_len), lambda i: (0, 0)),
        ],
        out_specs=pl.BlockSpec((BATCH_BLOCK, embed, seq_len), lambda i: (i, 0, 0)),
        out_shape=jax.ShapeDtypeStruct((batch, embed, seq_len), x.dtype),
    )(xt, tt)

    out_t = pl.pallas_call(
        _tc_add_tail_body,
        grid=(TAIL_BLOCKS,),
        in_specs=[
            pl.BlockSpec(memory_space=pl.ANY),
            pl.BlockSpec(
                (BATCH_BLOCK, embed, seq_len), lambda i: (i + head, 0, 0)
            ),
            pl.BlockSpec((embed, seq_len), lambda i: (0, 0)),
        ],
        out_specs=pl.BlockSpec(
            (BATCH_BLOCK, embed, seq_len), lambda i: (i + head, 0, 0)
        ),
        out_shape=jax.ShapeDtypeStruct((batch, embed, seq_len), x.dtype),
        input_output_aliases={0: 0},
    )(head_out, xt, pe_t)

    return jnp.transpose(out_t, (0, 2, 1))


# speedup vs baseline: 2.3069x; 1.0100x over previous
import jax
import jax.numpy as jnp
from jax import lax
from jax.experimental import pallas as pl
from jax.experimental.pallas import tpu as pltpu
from jax.experimental.pallas import tpu_sc as plsc


NC, NS = 2, 16
L = 16
BATCH_BLOCK = 64
TAIL_BLOCKS = 1


def _sc_gather_body(t_hbm, o_hbm, rows_v, sem):
    wid = lax.axis_index("s") * NC + lax.axis_index("c")
    n_pos = o_hbm.shape[0]
    active = n_pos // L

    @pl.when(wid < active)
    def _():
        for k in range(2):
            pltpu.async_copy(
                t_hbm.at[pl.ds((wid * 2 + k) * 8, 8)],
                rows_v.at[pl.ds(k * 8, 8)],
                sem,
            )
        for k in range(2):
            pltpu.make_async_copy(
                t_hbm.at[pl.ds((wid * 2 + k) * 8, 8)],
                rows_v.at[pl.ds(k * 8, 8)],
                sem,
            ).wait()
        pltpu.sync_copy(rows_v, o_hbm.at[pl.ds(wid * L, L)])


def _tc_add_body(x_ref, t_ref, o_ref):
    o_ref[...] = x_ref[...] + t_ref[...][None, :, :]


def _tc_add_tail_body(prev_ref, x_ref, t_ref, o_ref):
    del prev_ref
    o_ref[...] = x_ref[...] + jnp.transpose(t_ref[...], (1, 0))[None, :, :]


def kernel(x, pos_table):
    batch, seq_len, embed = x.shape
    nblocks = batch // BATCH_BLOCK
    head = nblocks - TAIL_BLOCKS

    mesh = plsc.VectorSubcoreMesh(core_axis_name="c", subcore_axis_name="s")
    pos_emb = pl.kernel(
        _sc_gather_body,
        out_type=jax.ShapeDtypeStruct((seq_len, embed), pos_table.dtype),
        mesh=mesh,
        scratch_types=[
            pltpu.VMEM((L, embed), jnp.float32),
            pltpu.SemaphoreType.DMA,
        ],
        compiler_params=pltpu.CompilerParams(use_tc_tiling_on_sc=True),
    )(pos_table)

    xt = jnp.transpose(x, (0, 2, 1))
    tt = jnp.transpose(pos_table[:seq_len], (1, 0))

    head_out = pl.pallas_call(
        _tc_add_body,
        grid=(head,),
        in_specs=[
            pl.BlockSpec((BATCH_BLOCK, embed, seq_len), lambda i: (i, 0, 0)),
            pl.BlockSpec((embed, seq_len), lambda i: (0, 0)),
        ],
        out_specs=pl.BlockSpec((BATCH_BLOCK, embed, seq_len), lambda i: (i, 0, 0)),
        out_shape=jax.ShapeDtypeStruct((batch, embed, seq_len), x.dtype),
    )(xt, tt)

    out_t = pl.pallas_call(
        _tc_add_tail_body,
        grid=(TAIL_BLOCKS,),
        in_specs=[
            pl.BlockSpec(memory_space=pl.ANY),
            pl.BlockSpec(
                (BATCH_BLOCK, embed, seq_len), lambda i: (i + head, 0, 0)
            ),
            pl.BlockSpec((seq_len, embed), lambda i: (0, 0)),
        ],
        out_specs=pl.BlockSpec(
            (BATCH_BLOCK, embed, seq_len), lambda i: (i + head, 0, 0)
        ),
        out_shape=jax.ShapeDtypeStruct((batch, embed, seq_len), x.dtype),
        input_output_aliases={0: 0},
    )(head_out, xt, pos_emb)

    return jnp.transpose(out_t, (0, 2, 1))
